# baseline (speedup 1.0000x reference)
import functools

import jax
import jax.numpy as jnp
from jax import lax
from jax.experimental import pallas as pl
from jax.experimental.pallas import tpu as pltpu
from jax.experimental.pallas import tpu_sc as plsc

F32 = jnp.float32
I32 = jnp.int32
D = 512
BM = 512
BLK = 128
CHK = 256
BIG = 2 ** 30

NC, NS = 2, 16
NW = NC * NS
G = 192
ALIGN = NW * G

_N = {"mention": 50000, "sentence": 10000, "entity": 10000}
_NPAD = {"mention": 50176, "sentence": 10240, "entity": 10240}
_TID = {"mention": 0, "sentence": 1, "entity": 2}

_ETS = {
    "mention_in_sentence_sentence": ("mention", "sentence"),
    "sentence_contains_mention": ("sentence", "mention"),
    "mention_coref_mention": ("mention", "mention"),
    "entity_has_mention_mention": ("entity", "mention"),
    "mention_belongs_to_entity": ("mention", "entity"),
    "sentence_adjacent_sentence": ("sentence", "sentence"),
    "mention_same_sent_mention": ("mention", "mention"),
    "entity_self_loop_entity": ("entity", "entity"),
    "mention_self_loop_mention": ("mention", "mention"),
}
_SRC_BLOCKS = {
    "mention": ["mention_in_sentence_sentence", "mention_coref_mention",
                "mention_belongs_to_entity", "mention_same_sent_mention",
                "mention_self_loop_mention"],
    "sentence": ["sentence_contains_mention", "sentence_adjacent_sentence"],
    "entity": ["entity_has_mention_mention", "entity_self_loop_entity"],
}
_POS = {et: i for nt in _SRC_BLOCKS for i, et in enumerate(_SRC_BLOCKS[nt])}
_DST_ETS = {
    nt: [et for et, (_, d) in _ETS.items() if d == nt] for nt in _N
}


def _ceil_to(x, m):
    return -(-x // m) * m


def _mm_body(comp_ref, x_ref, bases_ref, out_ref, w_ref):
    j = pl.program_id(0)
    i = pl.program_id(1)
    s_dim = bases_ref.shape[1]

    @pl.when(i == 0)
    def _():
        w = comp_ref[j, 0] * bases_ref[0, 0]
        for s in range(1, s_dim):
            w += comp_ref[j, s] * bases_ref[0, s]
        w_ref[...] = w

    out_ref[0] = jnp.dot(x_ref[...], w_ref[...], preferred_element_type=F32)


def _run_matmul(x_pad, bases_stack, comp_stack):
    ncb, s_dim = comp_stack.shape
    npad = x_pad.shape[0]
    mb = npad // BM
    return pl.pallas_call(
        _mm_body,
        grid=(ncb, mb),
        in_specs=[
            pl.BlockSpec(memory_space=pltpu.SMEM),
            pl.BlockSpec((BM, D), lambda j, i: (i, 0)),
            pl.BlockSpec((1, s_dim, D, D), lambda j, i: (j, 0, 0, 0)),
        ],
        out_specs=pl.BlockSpec((1, BM, D), lambda j, i: (j, i, 0)),
        out_shape=jax.ShapeDtypeStruct((ncb, npad, D), F32),
        scratch_shapes=[pltpu.VMEM((D, D), F32)],
    )(comp_stack, x_pad, bases_stack)


def _make_gather(streams, n_tables, out_rows):
    mesh = plsc.VectorSubcoreMesh(core_axis_name="c", subcore_axis_name="s")

    def body(*refs):
        tables = refs[:n_tables]
        idx_hbm, out_hbm, idx_v, rows_v, sem = refs[n_tables:]
        wid = lax.axis_index("s") * NC + lax.axis_index("c")
        for (tid, off, per_w, nch) in streams:
            tbl = tables[tid]
            base0 = off + wid * per_w

            @pl.loop(0, nch)
            def _(t):
                b = base0 + t * G
                pltpu.sync_copy(idx_hbm.at[pl.ds(b, G)], idx_v)
                pltpu.async_copy(tbl.at[idx_v], rows_v, sem).wait()
                pltpu.sync_copy(rows_v, out_hbm.at[pl.ds(b, G)])

    return pl.kernel(
        body,
        out_type=jax.ShapeDtypeStruct((out_rows, D), F32),
        mesh=mesh,
        scratch_types=[
            pltpu.VMEM((G,), I32),
            pltpu.VMEM((G, D), F32),
            pltpu.SemaphoreType.DMA,
        ],
    )


def _make_seg(nb, root_pos, t_static):
    def body(eb_ref, ec_ref, fb_ref, dst_ref, msgs_ref, root_ref, out_ref):
        c = pl.program_id(0)
        blk_base = eb_ref[c] * BLK
        rows = lax.broadcasted_iota(I32, (BLK, CHK), 0) + blk_base
        onehot = (rows == dst_ref[0]).astype(F32)
        contrib = jnp.dot(onehot, msgs_ref[...], preferred_element_type=F32)

        @pl.when(fb_ref[c] == 1)
        def _():
            out_ref[...] = root_ref[0] + contrib

        @pl.when(fb_ref[c] == 0)
        def _():
            out_ref[...] += contrib

    grid_spec = pltpu.PrefetchScalarGridSpec(
        num_scalar_prefetch=3,
        grid=(t_static,),
        in_specs=[
            pl.BlockSpec((1, 1, CHK), lambda c, eb, ec, fb: (ec[c], 0, 0)),
            pl.BlockSpec((CHK, D), lambda c, eb, ec, fb: (ec[c], 0)),
            pl.BlockSpec(
                (1, BLK, D),
                lambda c, eb, ec, fb: (root_pos, jnp.minimum(eb[c], nb - 1), 0),
            ),
        ],
        out_specs=pl.BlockSpec((BLK, D), lambda c, eb, ec, fb: (eb[c], 0)),
    )
    return pl.pallas_call(
        body,
        grid_spec=grid_spec,
        out_shape=jax.ShapeDtypeStruct(((nb + 4) * BLK, D), F32),
    )


def _chunk_tables(dst_sorted_list, offs, nb, t_static):
    ns = len(dst_sorted_list)
    cnts, fcs = [], []
    for s, (d, off) in enumerate(zip(dst_sorted_list, offs)):
        rp = jnp.searchsorted(
            d, jnp.arange(nb + 1, dtype=I32) * BLK, side="left"
        ).astype(I32)
        fc = (off + rp[:-1]) // CHK
        lc = (off + rp[1:] - 1) // CHK
        cnt = jnp.where(rp[1:] > rp[:-1], lc - fc + 1, 0)
        if s == 0:
            cnt = jnp.maximum(cnt, 1)
        cnts.append(cnt)
        fcs.append(fc)
    cnt = jnp.stack(cnts, axis=1).reshape(-1)
    fc = jnp.stack(fcs, axis=1).reshape(-1)
    csum = jnp.cumsum(cnt)
    starts = csum - cnt
    total = csum[-1]
    e = jnp.arange(t_static, dtype=I32)
    slot = jnp.searchsorted(csum, e, side="right").astype(I32)
    slot = jnp.minimum(slot, nb * ns - 1)
    j = e - starts[slot]
    valid = e < total
    b = slot // ns
    s_id = slot % ns
    eb = jnp.where(valid, b, nb).astype(I32)
    ec = jnp.where(valid, fc[slot] + j, 0).astype(I32)
    fb = (valid & (s_id == 0) & (j == 0)).astype(I32)
    return eb, ec, fb


def _ep_body(agg_ref, x_ref, bsum_ref, g_ref, b_ref, out_ref):
    h = jnp.maximum(agg_ref[...] + bsum_ref[...], 0.0) + x_ref[...]
    mu = jnp.mean(h, axis=-1, keepdims=True)
    var = jnp.mean((h - mu) * (h - mu), axis=-1, keepdims=True)
    out_ref[...] = (h - mu) * lax.rsqrt(var + 1e-5) * g_ref[...] + b_ref[...]


def _run_epilogue(agg_full, x_pad, bsum, gamma, beta):
    npad = x_pad.shape[0]
    grid = (npad // BM,)
    vec = lambda c, i: (0, 0)
    return pl.pallas_call(
        _ep_body,
        grid=grid,
        in_specs=[
            pl.BlockSpec((BM, D), lambda i: (i, 0)),
            pl.BlockSpec((BM, D), lambda i: (i, 0)),
            pl.BlockSpec((1, D), lambda i: (0, 0)),
            pl.BlockSpec((1, D), lambda i: (0, 0)),
            pl.BlockSpec((1, D), lambda i: (0, 0)),
        ],
        out_specs=pl.BlockSpec((BM, D), lambda i: (i, 0)),
        out_shape=jax.ShapeDtypeStruct((npad, D), F32),
    )(agg_full, x_pad, bsum.reshape(1, D), gamma.reshape(1, D),
      beta.reshape(1, D))


def kernel(x_mention, x_sentence, x_entity, ei_mention_in_sentence_sentence, bases_mention_in_sentence_sentence, comp_mention_in_sentence_sentence, root_mention_in_sentence_sentence, bias_mention_in_sentence_sentence, ei_sentence_contains_mention, bases_sentence_contains_mention, comp_sentence_contains_mention, root_sentence_contains_mention, bias_sentence_contains_mention, ei_mention_coref_mention, bases_mention_coref_mention, comp_mention_coref_mention, root_mention_coref_mention, bias_mention_coref_mention, ei_entity_has_mention_mention, bases_entity_has_mention_mention, comp_entity_has_mention_mention, root_entity_has_mention_mention, bias_entity_has_mention_mention, ei_mention_belongs_to_entity, bases_mention_belongs_to_entity, comp_mention_belongs_to_entity, root_mention_belongs_to_entity, bias_mention_belongs_to_entity, ei_sentence_adjacent_sentence, bases_sentence_adjacent_sentence, comp_sentence_adjacent_sentence, root_sentence_adjacent_sentence, bias_sentence_adjacent_sentence, ei_mention_same_sent_mention, bases_mention_same_sent_mention, comp_mention_same_sent_mention, root_mention_same_sent_mention, bias_mention_same_sent_mention, ei_entity_self_loop_entity, bases_entity_self_loop_entity, comp_entity_self_loop_entity, root_entity_self_loop_entity, bias_entity_self_loop_entity, ei_mention_self_loop_mention, bases_mention_self_loop_mention, comp_mention_self_loop_mention, root_mention_self_loop_mention, bias_mention_self_loop_mention, ln_gamma_mention, ln_beta_mention, ln_gamma_sentence, ln_beta_sentence, ln_gamma_entity, ln_beta_entity):
    inp = dict(locals())
    xs = {nt: inp["x_" + nt] for nt in _N}

    x_pad = {
        nt: jnp.pad(xs[nt], ((0, _NPAD[nt] - _N[nt]), (0, 0))) for nt in _N
    }

    tables = {}
    for nt in _N:
        ets = _SRC_BLOCKS[nt]
        roots = [inp["root_" + e] for e in _DST_ETS[nt]]
        s_dim = max(4, len(roots))
        b_rows, c_rows = [], []
        for e in ets:
            b = inp["bases_" + e]
            if s_dim > 4:
                b = jnp.concatenate(
                    [b, jnp.zeros((s_dim - 4, D, D), F32)], axis=0)
            b_rows.append(b)
            c = inp["comp_" + e].reshape(-1)
            c_rows.append(jnp.pad(c, (0, s_dim - 4)))
        rb = jnp.stack(roots, axis=0)
        if len(roots) < s_dim:
            rb = jnp.concatenate(
                [rb, jnp.zeros((s_dim - len(roots), D, D), F32)], axis=0)
        b_rows.append(rb)
        c_rows.append(
            jnp.pad(jnp.ones((len(roots),), F32), (0, s_dim - len(roots))))
        bases_stack = jnp.stack(b_rows, axis=0)
        comp_stack = jnp.stack(c_rows, axis=0)
        tables[nt] = _run_matmul(x_pad[nt], bases_stack, comp_stack)

    root_pos = {nt: len(_SRC_BLOCKS[nt]) for nt in _N}
    tables2d = {
        nt: tables[nt].reshape(-1, D) for nt in _N
    }

    agg = {}
    for nt in _N:
        dst_ets = _DST_ETS[nt]
        src_nts = sorted({_ETS[e][0] for e in dst_ets}, key=lambda n: _TID[n])
        idx_parts, dst_parts, streams, offs = [], [], [], []
        off = 0
        for slot, src_nt in enumerate(src_nts):
            rows_u, dsts_u = [], []
            for e in dst_ets:
                if _ETS[e][0] != src_nt:
                    continue
                ei = inp["ei_" + e]
                rows_u.append(_POS[e] * _NPAD[src_nt] + ei[0].astype(I32))
                dsts_u.append(ei[1].astype(I32))
            row_u = jnp.concatenate(rows_u)
            dst_u = jnp.concatenate(dsts_u)
            e_s = row_u.shape[0]
            e_pad = _ceil_to(e_s, ALIGN)
            dst_s, row_s = lax.sort_key_val(dst_u, row_u)
            pad_idx = (jnp.arange(e_pad - e_s, dtype=I32) * 8) % 30000
            idx_parts.append(jnp.concatenate([row_s, pad_idx]))
            dst_parts.append(jnp.pad(dst_s, (0, e_pad - e_s),
                                     constant_values=BIG))
            streams.append((slot, off, e_pad // NW, e_pad // ALIGN))
            offs.append(off)
            off += e_pad
        idx_all = jnp.concatenate(idx_parts)

        src_tables = [tables2d[n] for n in src_nts]
        gk = _make_gather(streams, len(src_tables), off)
        msgs = gk(*src_tables, idx_all)

        nb = _NPAD[nt] // BLK
        t_static = off // CHK + nb * len(src_nts)
        eb, ec, fb = _chunk_tables(dst_parts, offs, nb, t_static)
        dst3 = jnp.concatenate(dst_parts).reshape(-1, 1, CHK)
        seg = _make_seg(nb, root_pos[nt], t_static)
        agg[nt] = seg(eb, ec, fb, dst3, msgs, tables[nt])

    res = []
    for nt in _N:
        bsum = sum(inp["bias_" + e] for e in _DST_ETS[nt])
        out = _run_epilogue(agg[nt], x_pad[nt], bsum,
                            inp["ln_gamma_" + nt], inp["ln_beta_" + nt])
        res.append(out[: _N[nt]])
    return tuple(res)


# speedup vs baseline: 1.1501x; 1.1501x over previous
import functools

import jax
import jax.numpy as jnp
from jax import lax
from jax.experimental import pallas as pl
from jax.experimental.pallas import tpu as pltpu
from jax.experimental.pallas import tpu_sc as plsc

F32 = jnp.float32
I32 = jnp.int32
D = 512
BM = 512
BLK = 128
CHK = 1024
BIG = 2 ** 30

NC, NS = 2, 16
NW = NC * NS
G = 192
ALIGN = NW * G

_N = {"mention": 50000, "sentence": 10000, "entity": 10000}
_NPAD = {"mention": 50176, "sentence": 10240, "entity": 10240}
_TID = {"mention": 0, "sentence": 1, "entity": 2}

_ETS = {
    "mention_in_sentence_sentence": ("mention", "sentence"),
    "sentence_contains_mention": ("sentence", "mention"),
    "mention_coref_mention": ("mention", "mention"),
    "entity_has_mention_mention": ("entity", "mention"),
    "mention_belongs_to_entity": ("mention", "entity"),
    "sentence_adjacent_sentence": ("sentence", "sentence"),
    "mention_same_sent_mention": ("mention", "mention"),
    "entity_self_loop_entity": ("entity", "entity"),
    "mention_self_loop_mention": ("mention", "mention"),
}
_SRC_BLOCKS = {
    "mention": ["mention_in_sentence_sentence", "mention_coref_mention",
                "mention_belongs_to_entity", "mention_same_sent_mention",
                "mention_self_loop_mention"],
    "sentence": ["sentence_contains_mention", "sentence_adjacent_sentence"],
    "entity": ["entity_has_mention_mention", "entity_self_loop_entity"],
}
_POS = {et: i for nt in _SRC_BLOCKS for i, et in enumerate(_SRC_BLOCKS[nt])}
_DST_ETS = {
    nt: [et for et, (_, d) in _ETS.items() if d == nt] for nt in _N
}


def _ceil_to(x, m):
    return -(-x // m) * m


def _mm_body(comp_ref, x_ref, bases_ref, out_ref, w_ref):
    j = pl.program_id(0)
    i = pl.program_id(1)
    s_dim = bases_ref.shape[1]

    @pl.when(i == 0)
    def _():
        w = comp_ref[j, 0] * bases_ref[0, 0]
        for s in range(1, s_dim):
            w += comp_ref[j, s] * bases_ref[0, s]
        w_ref[...] = w

    out_ref[0] = jnp.dot(x_ref[...], w_ref[...], preferred_element_type=F32)


def _run_matmul(x_pad, bases_stack, comp_stack):
    ncb, s_dim = comp_stack.shape
    npad = x_pad.shape[0]
    mb = npad // BM
    return pl.pallas_call(
        _mm_body,
        grid=(ncb, mb),
        in_specs=[
            pl.BlockSpec(memory_space=pltpu.SMEM),
            pl.BlockSpec((BM, D), lambda j, i: (i, 0)),
            pl.BlockSpec((1, s_dim, D, D), lambda j, i: (j, 0, 0, 0)),
        ],
        out_specs=pl.BlockSpec((1, BM, D), lambda j, i: (j, i, 0)),
        out_shape=jax.ShapeDtypeStruct((ncb, npad, D), F32),
        scratch_shapes=[pltpu.VMEM((D, D), F32)],
    )(comp_stack, x_pad, bases_stack)


def _make_gather(streams, n_tables, out_rows):
    mesh = plsc.VectorSubcoreMesh(core_axis_name="c", subcore_axis_name="s")

    def body(*refs):
        tables = refs[:n_tables]
        idx_hbm, out_hbm, idx_v, rows_v, sem = refs[n_tables:]
        wid = lax.axis_index("s") * NC + lax.axis_index("c")
        for (tid, off, per_w, nch) in streams:
            tbl = tables[tid]
            base0 = off + wid * per_w

            @pl.loop(0, nch)
            def _(t):
                b = base0 + t * G
                pltpu.sync_copy(idx_hbm.at[pl.ds(b, G)], idx_v)
                pltpu.async_copy(tbl.at[idx_v], rows_v, sem).wait()
                pltpu.sync_copy(rows_v, out_hbm.at[pl.ds(b, G)])

    return pl.kernel(
        body,
        out_type=jax.ShapeDtypeStruct((out_rows, D), F32),
        mesh=mesh,
        scratch_types=[
            pltpu.VMEM((G,), I32),
            pltpu.VMEM((G, D), F32),
            pltpu.SemaphoreType.DMA,
        ],
    )


def _make_seg(nb, root_pos, t_static):
    def body(eb_ref, ec_ref, fb_ref, dst_ref, msgs_ref, root_ref, out_ref):
        c = pl.program_id(0)
        blk_base = eb_ref[c] * BLK
        rows = lax.broadcasted_iota(I32, (BLK, CHK), 0) + blk_base
        onehot = (rows == dst_ref[0]).astype(F32)
        contrib = jnp.dot(onehot, msgs_ref[...], preferred_element_type=F32)

        @pl.when(fb_ref[c] == 1)
        def _():
            out_ref[...] = root_ref[0] + contrib

        @pl.when(fb_ref[c] == 0)
        def _():
            out_ref[...] += contrib

    grid_spec = pltpu.PrefetchScalarGridSpec(
        num_scalar_prefetch=3,
        grid=(t_static,),
        in_specs=[
            pl.BlockSpec((1, 1, CHK), lambda c, eb, ec, fb: (ec[c], 0, 0)),
            pl.BlockSpec((CHK, D), lambda c, eb, ec, fb: (ec[c], 0)),
            pl.BlockSpec(
                (1, BLK, D),
                lambda c, eb, ec, fb: (root_pos, jnp.minimum(eb[c], nb - 1), 0),
            ),
        ],
        out_specs=pl.BlockSpec((BLK, D), lambda c, eb, ec, fb: (eb[c], 0)),
    )
    return pl.pallas_call(
        body,
        grid_spec=grid_spec,
        out_shape=jax.ShapeDtypeStruct(((nb + 4) * BLK, D), F32),
    )


def _chunk_tables(dst_sorted_list, offs, nb, t_static):
    ns = len(dst_sorted_list)
    cnts, fcs = [], []
    for s, (d, off) in enumerate(zip(dst_sorted_list, offs)):
        rp = jnp.searchsorted(
            d, jnp.arange(nb + 1, dtype=I32) * BLK, side="left"
        ).astype(I32)
        fc = (off + rp[:-1]) // CHK
        lc = (off + rp[1:] - 1) // CHK
        cnt = jnp.where(rp[1:] > rp[:-1], lc - fc + 1, 0)
        if s == 0:
            cnt = jnp.maximum(cnt, 1)
        cnts.append(cnt)
        fcs.append(fc)
    cnt = jnp.stack(cnts, axis=1).reshape(-1)
    fc = jnp.stack(fcs, axis=1).reshape(-1)
    csum = jnp.cumsum(cnt)
    starts = csum - cnt
    total = csum[-1]
    e = jnp.arange(t_static, dtype=I32)
    slot = jnp.searchsorted(csum, e, side="right").astype(I32)
    slot = jnp.minimum(slot, nb * ns - 1)
    j = e - starts[slot]
    valid = e < total
    b = slot // ns
    s_id = slot % ns
    eb = jnp.where(valid, b, nb).astype(I32)
    ec = jnp.where(valid, fc[slot] + j, 0).astype(I32)
    fb = (valid & (s_id == 0) & (j == 0)).astype(I32)
    return eb, ec, fb


def _ep_body(agg_ref, x_ref, bsum_ref, g_ref, b_ref, out_ref):
    h = jnp.maximum(agg_ref[...] + bsum_ref[...], 0.0) + x_ref[...]
    mu = jnp.mean(h, axis=-1, keepdims=True)
    var = jnp.mean((h - mu) * (h - mu), axis=-1, keepdims=True)
    out_ref[...] = (h - mu) * lax.rsqrt(var + 1e-5) * g_ref[...] + b_ref[...]


def _run_epilogue(agg_full, x_pad, bsum, gamma, beta):
    npad = x_pad.shape[0]
    grid = (npad // BM,)
    vec = lambda c, i: (0, 0)
    return pl.pallas_call(
        _ep_body,
        grid=grid,
        in_specs=[
            pl.BlockSpec((BM, D), lambda i: (i, 0)),
            pl.BlockSpec((BM, D), lambda i: (i, 0)),
            pl.BlockSpec((1, D), lambda i: (0, 0)),
            pl.BlockSpec((1, D), lambda i: (0, 0)),
            pl.BlockSpec((1, D), lambda i: (0, 0)),
        ],
        out_specs=pl.BlockSpec((BM, D), lambda i: (i, 0)),
        out_shape=jax.ShapeDtypeStruct((npad, D), F32),
    )(agg_full, x_pad, bsum.reshape(1, D), gamma.reshape(1, D),
      beta.reshape(1, D))


def kernel(x_mention, x_sentence, x_entity, ei_mention_in_sentence_sentence, bases_mention_in_sentence_sentence, comp_mention_in_sentence_sentence, root_mention_in_sentence_sentence, bias_mention_in_sentence_sentence, ei_sentence_contains_mention, bases_sentence_contains_mention, comp_sentence_contains_mention, root_sentence_contains_mention, bias_sentence_contains_mention, ei_mention_coref_mention, bases_mention_coref_mention, comp_mention_coref_mention, root_mention_coref_mention, bias_mention_coref_mention, ei_entity_has_mention_mention, bases_entity_has_mention_mention, comp_entity_has_mention_mention, root_entity_has_mention_mention, bias_entity_has_mention_mention, ei_mention_belongs_to_entity, bases_mention_belongs_to_entity, comp_mention_belongs_to_entity, root_mention_belongs_to_entity, bias_mention_belongs_to_entity, ei_sentence_adjacent_sentence, bases_sentence_adjacent_sentence, comp_sentence_adjacent_sentence, root_sentence_adjacent_sentence, bias_sentence_adjacent_sentence, ei_mention_same_sent_mention, bases_mention_same_sent_mention, comp_mention_same_sent_mention, root_mention_same_sent_mention, bias_mention_same_sent_mention, ei_entity_self_loop_entity, bases_entity_self_loop_entity, comp_entity_self_loop_entity, root_entity_self_loop_entity, bias_entity_self_loop_entity, ei_mention_self_loop_mention, bases_mention_self_loop_mention, comp_mention_self_loop_mention, root_mention_self_loop_mention, bias_mention_self_loop_mention, ln_gamma_mention, ln_beta_mention, ln_gamma_sentence, ln_beta_sentence, ln_gamma_entity, ln_beta_entity):
    inp = dict(locals())
    xs = {nt: inp["x_" + nt] for nt in _N}

    x_pad = {
        nt: jnp.pad(xs[nt], ((0, _NPAD[nt] - _N[nt]), (0, 0))) for nt in _N
    }

    tables = {}
    for nt in _N:
        ets = _SRC_BLOCKS[nt]
        roots = [inp["root_" + e] for e in _DST_ETS[nt]]
        s_dim = max(4, len(roots))
        b_rows, c_rows = [], []
        for e in ets:
            b = inp["bases_" + e]
            if s_dim > 4:
                b = jnp.concatenate(
                    [b, jnp.zeros((s_dim - 4, D, D), F32)], axis=0)
            b_rows.append(b)
            c = inp["comp_" + e].reshape(-1)
            c_rows.append(jnp.pad(c, (0, s_dim - 4)))
        rb = jnp.stack(roots, axis=0)
        if len(roots) < s_dim:
            rb = jnp.concatenate(
                [rb, jnp.zeros((s_dim - len(roots), D, D), F32)], axis=0)
        b_rows.append(rb)
        c_rows.append(
            jnp.pad(jnp.ones((len(roots),), F32), (0, s_dim - len(roots))))
        bases_stack = jnp.stack(b_rows, axis=0)
        comp_stack = jnp.stack(c_rows, axis=0)
        tables[nt] = _run_matmul(x_pad[nt], bases_stack, comp_stack)

    root_pos = {nt: len(_SRC_BLOCKS[nt]) for nt in _N}
    tables2d = {
        nt: tables[nt].reshape(-1, D) for nt in _N
    }

    agg = {}
    for nt in _N:
        dst_ets = _DST_ETS[nt]
        src_nts = sorted({_ETS[e][0] for e in dst_ets}, key=lambda n: _TID[n])
        idx_parts, dst_parts, streams, offs = [], [], [], []
        off = 0
        for slot, src_nt in enumerate(src_nts):
            rows_u, dsts_u = [], []
            for e in dst_ets:
                if _ETS[e][0] != src_nt:
                    continue
                ei = inp["ei_" + e]
                rows_u.append(_POS[e] * _NPAD[src_nt] + ei[0].astype(I32))
                dsts_u.append(ei[1].astype(I32))
            row_u = jnp.concatenate(rows_u)
            dst_u = jnp.concatenate(dsts_u)
            e_s = row_u.shape[0]
            e_pad = _ceil_to(e_s, ALIGN)
            dst_s, row_s = lax.sort_key_val(dst_u, row_u)
            pad_idx = (jnp.arange(e_pad - e_s, dtype=I32) * 8) % 30000
            idx_parts.append(jnp.concatenate([row_s, pad_idx]))
            dst_parts.append(jnp.pad(dst_s, (0, e_pad - e_s),
                                     constant_values=BIG))
            streams.append((slot, off, e_pad // NW, e_pad // ALIGN))
            offs.append(off)
            off += e_pad
        idx_all = jnp.concatenate(idx_parts)

        src_tables = [tables2d[n] for n in src_nts]
        gk = _make_gather(streams, len(src_tables), off)
        msgs = gk(*src_tables, idx_all)

        nb = _NPAD[nt] // BLK
        t_static = off // CHK + nb * len(src_nts)
        eb, ec, fb = _chunk_tables(dst_parts, offs, nb, t_static)
        dst3 = jnp.concatenate(dst_parts).reshape(-1, 1, CHK)
        seg = _make_seg(nb, root_pos[nt], t_static)
        agg[nt] = seg(eb, ec, fb, dst3, msgs, tables[nt])

    res = []
    for nt in _N:
        bsum = sum(inp["bias_" + e] for e in _DST_ETS[nt])
        out = _run_epilogue(agg[nt], x_pad[nt], bsum,
                            inp["ln_gamma_" + nt], inp["ln_beta_" + nt])
        res.append(out[: _N[nt]])
    return tuple(res)


# speedup vs baseline: 1.3442x; 1.1687x over previous
import functools

import jax
import jax.numpy as jnp
from jax import lax
from jax.experimental import pallas as pl
from jax.experimental.pallas import tpu as pltpu
from jax.experimental.pallas import tpu_sc as plsc

F32 = jnp.float32
I32 = jnp.int32
D = 512
BM = 512
BLK = 256
CHK = 1024
BIG = 2 ** 30

NC, NS = 2, 16
NW = NC * NS
G = 192
ALIGN = NW * G

_N = {"mention": 50000, "sentence": 10000, "entity": 10000}
_NPAD = {"mention": 50176, "sentence": 10240, "entity": 10240}
_TID = {"mention": 0, "sentence": 1, "entity": 2}

_ETS = {
    "mention_in_sentence_sentence": ("mention", "sentence"),
    "sentence_contains_mention": ("sentence", "mention"),
    "mention_coref_mention": ("mention", "mention"),
    "entity_has_mention_mention": ("entity", "mention"),
    "mention_belongs_to_entity": ("mention", "entity"),
    "sentence_adjacent_sentence": ("sentence", "sentence"),
    "mention_same_sent_mention": ("mention", "mention"),
    "entity_self_loop_entity": ("entity", "entity"),
    "mention_self_loop_mention": ("mention", "mention"),
}
_SRC_BLOCKS = {
    "mention": ["mention_in_sentence_sentence", "mention_coref_mention",
                "mention_belongs_to_entity", "mention_same_sent_mention",
                "mention_self_loop_mention"],
    "sentence": ["sentence_contains_mention", "sentence_adjacent_sentence"],
    "entity": ["entity_has_mention_mention", "entity_self_loop_entity"],
}
_POS = {et: i for nt in _SRC_BLOCKS for i, et in enumerate(_SRC_BLOCKS[nt])}
_DST_ETS = {
    nt: [et for et, (_, d) in _ETS.items() if d == nt] for nt in _N
}


def _ceil_to(x, m):
    return -(-x // m) * m


def _mm_body(comp_ref, x_ref, bases_ref, out_ref, w_ref):
    j = pl.program_id(0)
    i = pl.program_id(1)
    s_dim = bases_ref.shape[1]

    @pl.when(i == 0)
    def _():
        w = comp_ref[j, 0] * bases_ref[0, 0]
        for s in range(1, s_dim):
            w += comp_ref[j, s] * bases_ref[0, s]
        w_ref[...] = w

    out_ref[0] = jnp.dot(x_ref[...], w_ref[...], preferred_element_type=F32)


def _run_matmul(x_pad, bases_stack, comp_stack):
    ncb, s_dim = comp_stack.shape
    npad = x_pad.shape[0]
    mb = npad // BM
    return pl.pallas_call(
        _mm_body,
        grid=(ncb, mb),
        in_specs=[
            pl.BlockSpec(memory_space=pltpu.SMEM),
            pl.BlockSpec((BM, D), lambda j, i: (i, 0)),
            pl.BlockSpec((1, s_dim, D, D), lambda j, i: (j, 0, 0, 0)),
        ],
        out_specs=pl.BlockSpec((1, BM, D), lambda j, i: (j, i, 0)),
        out_shape=jax.ShapeDtypeStruct((ncb, npad, D), F32),
        scratch_shapes=[pltpu.VMEM((D, D), F32)],
    )(comp_stack, x_pad, bases_stack)


def _make_gather(streams, n_tables, out_rows):
    mesh = plsc.VectorSubcoreMesh(core_axis_name="c", subcore_axis_name="s")

    def body(*refs):
        tables = refs[:n_tables]
        idx_hbm, out_hbm, idx_v, rows_v, sem = refs[n_tables:]
        wid = lax.axis_index("s") * NC + lax.axis_index("c")
        for (tid, off, per_w, nch) in streams:
            tbl = tables[tid]
            base0 = off + wid * per_w

            @pl.loop(0, nch)
            def _(t):
                b = base0 + t * G
                pltpu.sync_copy(idx_hbm.at[pl.ds(b, G)], idx_v)
                pltpu.async_copy(tbl.at[idx_v], rows_v, sem).wait()
                pltpu.sync_copy(rows_v, out_hbm.at[pl.ds(b, G)])

    return pl.kernel(
        body,
        out_type=jax.ShapeDtypeStruct((out_rows, D), F32),
        mesh=mesh,
        scratch_types=[
            pltpu.VMEM((G,), I32),
            pltpu.VMEM((G, D), F32),
            pltpu.SemaphoreType.DMA,
        ],
    )


def _make_seg(nb, root_pos, t_static):
    def body(eb_ref, ec_ref, fb_ref, dst_ref, msgs_ref, root_ref, out_ref):
        c = pl.program_id(0)
        blk_base = eb_ref[c] * BLK
        rows = lax.broadcasted_iota(I32, (BLK, CHK), 0) + blk_base
        onehot = (rows == dst_ref[0]).astype(F32)
        contrib = jnp.dot(onehot, msgs_ref[...], preferred_element_type=F32)

        @pl.when(fb_ref[c] == 1)
        def _():
            out_ref[...] = root_ref[0] + contrib

        @pl.when(fb_ref[c] == 0)
        def _():
            out_ref[...] += contrib

    grid_spec = pltpu.PrefetchScalarGridSpec(
        num_scalar_prefetch=3,
        grid=(t_static,),
        in_specs=[
            pl.BlockSpec((1, 1, CHK), lambda c, eb, ec, fb: (ec[c], 0, 0)),
            pl.BlockSpec((CHK, D), lambda c, eb, ec, fb: (ec[c], 0)),
            pl.BlockSpec(
                (1, BLK, D),
                lambda c, eb, ec, fb: (root_pos, jnp.minimum(eb[c], nb - 1), 0),
            ),
        ],
        out_specs=pl.BlockSpec((BLK, D), lambda c, eb, ec, fb: (eb[c], 0)),
    )
    return pl.pallas_call(
        body,
        grid_spec=grid_spec,
        out_shape=jax.ShapeDtypeStruct(((nb + 4) * BLK, D), F32),
    )


def _chunk_tables(dst_sorted_list, offs, nb, t_static):
    ns = len(dst_sorted_list)
    cnts, fcs = [], []
    for s, (d, off) in enumerate(zip(dst_sorted_list, offs)):
        rp = jnp.searchsorted(
            d, jnp.arange(nb + 1, dtype=I32) * BLK, side="left"
        ).astype(I32)
        fc = (off + rp[:-1]) // CHK
        lc = (off + rp[1:] - 1) // CHK
        cnt = jnp.where(rp[1:] > rp[:-1], lc - fc + 1, 0)
        if s == 0:
            cnt = jnp.maximum(cnt, 1)
        cnts.append(cnt)
        fcs.append(fc)
    cnt = jnp.stack(cnts, axis=1).reshape(-1)
    fc = jnp.stack(fcs, axis=1).reshape(-1)
    csum = jnp.cumsum(cnt)
    starts = csum - cnt
    total = csum[-1]
    e = jnp.arange(t_static, dtype=I32)
    slot = jnp.searchsorted(csum, e, side="right").astype(I32)
    slot = jnp.minimum(slot, nb * ns - 1)
    j = e - starts[slot]
    valid = e < total
    b = slot // ns
    s_id = slot % ns
    eb = jnp.where(valid, b, nb).astype(I32)
    ec = jnp.where(valid, fc[slot] + j, 0).astype(I32)
    fb = (valid & (s_id == 0) & (j == 0)).astype(I32)
    return eb, ec, fb


def _ep_body(agg_ref, x_ref, bsum_ref, g_ref, b_ref, out_ref):
    h = jnp.maximum(agg_ref[...] + bsum_ref[...], 0.0) + x_ref[...]
    mu = jnp.mean(h, axis=-1, keepdims=True)
    var = jnp.mean((h - mu) * (h - mu), axis=-1, keepdims=True)
    out_ref[...] = (h - mu) * lax.rsqrt(var + 1e-5) * g_ref[...] + b_ref[...]


def _run_epilogue(agg_full, x_pad, bsum, gamma, beta):
    npad = x_pad.shape[0]
    grid = (npad // BM,)
    vec = lambda c, i: (0, 0)
    return pl.pallas_call(
        _ep_body,
        grid=grid,
        in_specs=[
            pl.BlockSpec((BM, D), lambda i: (i, 0)),
            pl.BlockSpec((BM, D), lambda i: (i, 0)),
            pl.BlockSpec((1, D), lambda i: (0, 0)),
            pl.BlockSpec((1, D), lambda i: (0, 0)),
            pl.BlockSpec((1, D), lambda i: (0, 0)),
        ],
        out_specs=pl.BlockSpec((BM, D), lambda i: (i, 0)),
        out_shape=jax.ShapeDtypeStruct((npad, D), F32),
    )(agg_full, x_pad, bsum.reshape(1, D), gamma.reshape(1, D),
      beta.reshape(1, D))


def kernel(x_mention, x_sentence, x_entity, ei_mention_in_sentence_sentence, bases_mention_in_sentence_sentence, comp_mention_in_sentence_sentence, root_mention_in_sentence_sentence, bias_mention_in_sentence_sentence, ei_sentence_contains_mention, bases_sentence_contains_mention, comp_sentence_contains_mention, root_sentence_contains_mention, bias_sentence_contains_mention, ei_mention_coref_mention, bases_mention_coref_mention, comp_mention_coref_mention, root_mention_coref_mention, bias_mention_coref_mention, ei_entity_has_mention_mention, bases_entity_has_mention_mention, comp_entity_has_mention_mention, root_entity_has_mention_mention, bias_entity_has_mention_mention, ei_mention_belongs_to_entity, bases_mention_belongs_to_entity, comp_mention_belongs_to_entity, root_mention_belongs_to_entity, bias_mention_belongs_to_entity, ei_sentence_adjacent_sentence, bases_sentence_adjacent_sentence, comp_sentence_adjacent_sentence, root_sentence_adjacent_sentence, bias_sentence_adjacent_sentence, ei_mention_same_sent_mention, bases_mention_same_sent_mention, comp_mention_same_sent_mention, root_mention_same_sent_mention, bias_mention_same_sent_mention, ei_entity_self_loop_entity, bases_entity_self_loop_entity, comp_entity_self_loop_entity, root_entity_self_loop_entity, bias_entity_self_loop_entity, ei_mention_self_loop_mention, bases_mention_self_loop_mention, comp_mention_self_loop_mention, root_mention_self_loop_mention, bias_mention_self_loop_mention, ln_gamma_mention, ln_beta_mention, ln_gamma_sentence, ln_beta_sentence, ln_gamma_entity, ln_beta_entity):
    inp = dict(locals())
    xs = {nt: inp["x_" + nt] for nt in _N}

    x_pad = {
        nt: jnp.pad(xs[nt], ((0, _NPAD[nt] - _N[nt]), (0, 0))) for nt in _N
    }

    tables = {}
    for nt in _N:
        ets = _SRC_BLOCKS[nt]
        roots = [inp["root_" + e] for e in _DST_ETS[nt]]
        s_dim = max(4, len(roots))
        b_rows, c_rows = [], []
        for e in ets:
            b = inp["bases_" + e]
            if s_dim > 4:
                b = jnp.concatenate(
                    [b, jnp.zeros((s_dim - 4, D, D), F32)], axis=0)
            b_rows.append(b)
            c = inp["comp_" + e].reshape(-1)
            c_rows.append(jnp.pad(c, (0, s_dim - 4)))
        rb = jnp.stack(roots, axis=0)
        if len(roots) < s_dim:
            rb = jnp.concatenate(
                [rb, jnp.zeros((s_dim - len(roots), D, D), F32)], axis=0)
        b_rows.append(rb)
        c_rows.append(
            jnp.pad(jnp.ones((len(roots),), F32), (0, s_dim - len(roots))))
        bases_stack = jnp.stack(b_rows, axis=0)
        comp_stack = jnp.stack(c_rows, axis=0)
        tables[nt] = _run_matmul(x_pad[nt], bases_stack, comp_stack)

    root_pos = {nt: len(_SRC_BLOCKS[nt]) for nt in _N}
    tables2d = {
        nt: tables[nt].reshape(-1, D) for nt in _N
    }

    agg = {}
    for nt in _N:
        dst_ets = _DST_ETS[nt]
        src_nts = sorted({_ETS[e][0] for e in dst_ets}, key=lambda n: _TID[n])
        idx_parts, dst_parts, streams, offs = [], [], [], []
        off = 0
        for slot, src_nt in enumerate(src_nts):
            rows_u, dsts_u = [], []
            for e in dst_ets:
                if _ETS[e][0] != src_nt:
                    continue
                ei = inp["ei_" + e]
                rows_u.append(_POS[e] * _NPAD[src_nt] + ei[0].astype(I32))
                dsts_u.append(ei[1].astype(I32))
            row_u = jnp.concatenate(rows_u)
            dst_u = jnp.concatenate(dsts_u)
            e_s = row_u.shape[0]
            e_pad = _ceil_to(e_s, ALIGN)
            dst_s, row_s = lax.sort_key_val(dst_u, row_u)
            pad_idx = (jnp.arange(e_pad - e_s, dtype=I32) * 8) % 30000
            idx_parts.append(jnp.concatenate([row_s, pad_idx]))
            dst_parts.append(jnp.pad(dst_s, (0, e_pad - e_s),
                                     constant_values=BIG))
            streams.append((slot, off, e_pad // NW, e_pad // ALIGN))
            offs.append(off)
            off += e_pad
        idx_all = jnp.concatenate(idx_parts)

        src_tables = [tables2d[n] for n in src_nts]
        gk = _make_gather(streams, len(src_tables), off)
        msgs = gk(*src_tables, idx_all)

        nb = _NPAD[nt] // BLK
        t_static = off // CHK + nb * len(src_nts)
        eb, ec, fb = _chunk_tables(dst_parts, offs, nb, t_static)
        dst3 = jnp.concatenate(dst_parts).reshape(-1, 1, CHK)
        seg = _make_seg(nb, root_pos[nt], t_static)
        agg[nt] = seg(eb, ec, fb, dst3, msgs, tables[nt])

    res = []
    for nt in _N:
        bsum = sum(inp["bias_" + e] for e in _DST_ETS[nt])
        out = _run_epilogue(agg[nt], x_pad[nt], bsum,
                            inp["ln_gamma_" + nt], inp["ln_beta_" + nt])
        res.append(out[: _N[nt]])
    return tuple(res)


# speedup vs baseline: 1.4088x; 1.0481x over previous
import functools

import jax
import jax.numpy as jnp
from jax import lax
from jax.experimental import pallas as pl
from jax.experimental.pallas import tpu as pltpu
from jax.experimental.pallas import tpu_sc as plsc

F32 = jnp.float32
I32 = jnp.int32
D = 512
BM = 512
BLK = 512
CHK = 1024
BIG = 2 ** 30

NC, NS = 2, 16
NW = NC * NS
G = 192
ALIGN = NW * G

_N = {"mention": 50000, "sentence": 10000, "entity": 10000}
_NPAD = {"mention": 50176, "sentence": 10240, "entity": 10240}
_TID = {"mention": 0, "sentence": 1, "entity": 2}

_ETS = {
    "mention_in_sentence_sentence": ("mention", "sentence"),
    "sentence_contains_mention": ("sentence", "mention"),
    "mention_coref_mention": ("mention", "mention"),
    "entity_has_mention_mention": ("entity", "mention"),
    "mention_belongs_to_entity": ("mention", "entity"),
    "sentence_adjacent_sentence": ("sentence", "sentence"),
    "mention_same_sent_mention": ("mention", "mention"),
    "entity_self_loop_entity": ("entity", "entity"),
    "mention_self_loop_mention": ("mention", "mention"),
}
_SRC_BLOCKS = {
    "mention": ["mention_in_sentence_sentence", "mention_coref_mention",
                "mention_belongs_to_entity", "mention_same_sent_mention",
                "mention_self_loop_mention"],
    "sentence": ["sentence_contains_mention", "sentence_adjacent_sentence"],
    "entity": ["entity_has_mention_mention", "entity_self_loop_entity"],
}
_POS = {et: i for nt in _SRC_BLOCKS for i, et in enumerate(_SRC_BLOCKS[nt])}
_DST_ETS = {
    nt: [et for et, (_, d) in _ETS.items() if d == nt] for nt in _N
}


def _ceil_to(x, m):
    return -(-x // m) * m


def _mm_body(comp_ref, x_ref, bases_ref, out_ref, w_ref):
    j = pl.program_id(0)
    i = pl.program_id(1)
    s_dim = bases_ref.shape[1]

    @pl.when(i == 0)
    def _():
        w = comp_ref[j, 0] * bases_ref[0, 0]
        for s in range(1, s_dim):
            w += comp_ref[j, s] * bases_ref[0, s]
        w_ref[...] = w

    out_ref[0] = jnp.dot(x_ref[...], w_ref[...], preferred_element_type=F32)


def _run_matmul(x_pad, bases_stack, comp_stack):
    ncb, s_dim = comp_stack.shape
    npad = x_pad.shape[0]
    mb = npad // BM
    return pl.pallas_call(
        _mm_body,
        grid=(ncb, mb),
        in_specs=[
            pl.BlockSpec(memory_space=pltpu.SMEM),
            pl.BlockSpec((BM, D), lambda j, i: (i, 0)),
            pl.BlockSpec((1, s_dim, D, D), lambda j, i: (j, 0, 0, 0)),
        ],
        out_specs=pl.BlockSpec((1, BM, D), lambda j, i: (j, i, 0)),
        out_shape=jax.ShapeDtypeStruct((ncb, npad, D), F32),
        scratch_shapes=[pltpu.VMEM((D, D), F32)],
    )(comp_stack, x_pad, bases_stack)


def _make_gather(streams, n_tables, out_rows):
    mesh = plsc.VectorSubcoreMesh(core_axis_name="c", subcore_axis_name="s")

    def body(*refs):
        tables = refs[:n_tables]
        idx_hbm, out_hbm, idx_v, rows_v, sem = refs[n_tables:]
        wid = lax.axis_index("s") * NC + lax.axis_index("c")
        for (tid, off, per_w, nch) in streams:
            tbl = tables[tid]
            base0 = off + wid * per_w

            @pl.loop(0, nch)
            def _(t):
                b = base0 + t * G
                pltpu.sync_copy(idx_hbm.at[pl.ds(b, G)], idx_v)
                pltpu.async_copy(tbl.at[idx_v], rows_v, sem).wait()
                pltpu.sync_copy(rows_v, out_hbm.at[pl.ds(b, G)])

    return pl.kernel(
        body,
        out_type=jax.ShapeDtypeStruct((out_rows, D), F32),
        mesh=mesh,
        scratch_types=[
            pltpu.VMEM((G,), I32),
            pltpu.VMEM((G, D), F32),
            pltpu.SemaphoreType.DMA,
        ],
    )


def _make_seg(nb, root_pos, t_static):
    def body(eb_ref, ec_ref, fb_ref, dst_ref, msgs_ref, root_ref, out_ref):
        c = pl.program_id(0)
        blk_base = eb_ref[c] * BLK
        rows = lax.broadcasted_iota(I32, (BLK, CHK), 0) + blk_base
        onehot = (rows == dst_ref[0]).astype(F32)
        contrib = jnp.dot(onehot, msgs_ref[...], preferred_element_type=F32)

        @pl.when(fb_ref[c] == 1)
        def _():
            out_ref[...] = root_ref[0] + contrib

        @pl.when(fb_ref[c] == 0)
        def _():
            out_ref[...] += contrib

    grid_spec = pltpu.PrefetchScalarGridSpec(
        num_scalar_prefetch=3,
        grid=(t_static,),
        in_specs=[
            pl.BlockSpec((1, 1, CHK), lambda c, eb, ec, fb: (ec[c], 0, 0)),
            pl.BlockSpec((CHK, D), lambda c, eb, ec, fb: (ec[c], 0)),
            pl.BlockSpec(
                (1, BLK, D),
                lambda c, eb, ec, fb: (root_pos, jnp.minimum(eb[c], nb - 1), 0),
            ),
        ],
        out_specs=pl.BlockSpec((BLK, D), lambda c, eb, ec, fb: (eb[c], 0)),
    )
    return pl.pallas_call(
        body,
        grid_spec=grid_spec,
        out_shape=jax.ShapeDtypeStruct(((nb + 4) * BLK, D), F32),
    )


def _chunk_tables(dst_sorted_list, offs, nb, t_static):
    ns = len(dst_sorted_list)
    cnts, fcs = [], []
    for s, (d, off) in enumerate(zip(dst_sorted_list, offs)):
        rp = jnp.searchsorted(
            d, jnp.arange(nb + 1, dtype=I32) * BLK, side="left"
        ).astype(I32)
        fc = (off + rp[:-1]) // CHK
        lc = (off + rp[1:] - 1) // CHK
        cnt = jnp.where(rp[1:] > rp[:-1], lc - fc + 1, 0)
        if s == 0:
            cnt = jnp.maximum(cnt, 1)
        cnts.append(cnt)
        fcs.append(fc)
    cnt = jnp.stack(cnts, axis=1).reshape(-1)
    fc = jnp.stack(fcs, axis=1).reshape(-1)
    csum = jnp.cumsum(cnt)
    starts = csum - cnt
    total = csum[-1]
    e = jnp.arange(t_static, dtype=I32)
    slot = jnp.searchsorted(csum, e, side="right").astype(I32)
    slot = jnp.minimum(slot, nb * ns - 1)
    j = e - starts[slot]
    valid = e < total
    b = slot // ns
    s_id = slot % ns
    eb = jnp.where(valid, b, nb).astype(I32)
    ec = jnp.where(valid, fc[slot] + j, 0).astype(I32)
    fb = (valid & (s_id == 0) & (j == 0)).astype(I32)
    return eb, ec, fb


def _ep_body(agg_ref, x_ref, bsum_ref, g_ref, b_ref, out_ref):
    h = jnp.maximum(agg_ref[...] + bsum_ref[...], 0.0) + x_ref[...]
    mu = jnp.mean(h, axis=-1, keepdims=True)
    var = jnp.mean((h - mu) * (h - mu), axis=-1, keepdims=True)
    out_ref[...] = (h - mu) * lax.rsqrt(var + 1e-5) * g_ref[...] + b_ref[...]


def _run_epilogue(agg_full, x_pad, bsum, gamma, beta):
    npad = x_pad.shape[0]
    grid = (npad // BM,)
    vec = lambda c, i: (0, 0)
    return pl.pallas_call(
        _ep_body,
        grid=grid,
        in_specs=[
            pl.BlockSpec((BM, D), lambda i: (i, 0)),
            pl.BlockSpec((BM, D), lambda i: (i, 0)),
            pl.BlockSpec((1, D), lambda i: (0, 0)),
            pl.BlockSpec((1, D), lambda i: (0, 0)),
            pl.BlockSpec((1, D), lambda i: (0, 0)),
        ],
        out_specs=pl.BlockSpec((BM, D), lambda i: (i, 0)),
        out_shape=jax.ShapeDtypeStruct((npad, D), F32),
    )(agg_full, x_pad, bsum.reshape(1, D), gamma.reshape(1, D),
      beta.reshape(1, D))


def kernel(x_mention, x_sentence, x_entity, ei_mention_in_sentence_sentence, bases_mention_in_sentence_sentence, comp_mention_in_sentence_sentence, root_mention_in_sentence_sentence, bias_mention_in_sentence_sentence, ei_sentence_contains_mention, bases_sentence_contains_mention, comp_sentence_contains_mention, root_sentence_contains_mention, bias_sentence_contains_mention, ei_mention_coref_mention, bases_mention_coref_mention, comp_mention_coref_mention, root_mention_coref_mention, bias_mention_coref_mention, ei_entity_has_mention_mention, bases_entity_has_mention_mention, comp_entity_has_mention_mention, root_entity_has_mention_mention, bias_entity_has_mention_mention, ei_mention_belongs_to_entity, bases_mention_belongs_to_entity, comp_mention_belongs_to_entity, root_mention_belongs_to_entity, bias_mention_belongs_to_entity, ei_sentence_adjacent_sentence, bases_sentence_adjacent_sentence, comp_sentence_adjacent_sentence, root_sentence_adjacent_sentence, bias_sentence_adjacent_sentence, ei_mention_same_sent_mention, bases_mention_same_sent_mention, comp_mention_same_sent_mention, root_mention_same_sent_mention, bias_mention_same_sent_mention, ei_entity_self_loop_entity, bases_entity_self_loop_entity, comp_entity_self_loop_entity, root_entity_self_loop_entity, bias_entity_self_loop_entity, ei_mention_self_loop_mention, bases_mention_self_loop_mention, comp_mention_self_loop_mention, root_mention_self_loop_mention, bias_mention_self_loop_mention, ln_gamma_mention, ln_beta_mention, ln_gamma_sentence, ln_beta_sentence, ln_gamma_entity, ln_beta_entity):
    inp = dict(locals())
    xs = {nt: inp["x_" + nt] for nt in _N}

    x_pad = {
        nt: jnp.pad(xs[nt], ((0, _NPAD[nt] - _N[nt]), (0, 0))) for nt in _N
    }

    tables = {}
    for nt in _N:
        ets = _SRC_BLOCKS[nt]
        roots = [inp["root_" + e] for e in _DST_ETS[nt]]
        s_dim = max(4, len(roots))
        b_rows, c_rows = [], []
        for e in ets:
            b = inp["bases_" + e]
            if s_dim > 4:
                b = jnp.concatenate(
                    [b, jnp.zeros((s_dim - 4, D, D), F32)], axis=0)
            b_rows.append(b)
            c = inp["comp_" + e].reshape(-1)
            c_rows.append(jnp.pad(c, (0, s_dim - 4)))
        rb = jnp.stack(roots, axis=0)
        if len(roots) < s_dim:
            rb = jnp.concatenate(
                [rb, jnp.zeros((s_dim - len(roots), D, D), F32)], axis=0)
        b_rows.append(rb)
        c_rows.append(
            jnp.pad(jnp.ones((len(roots),), F32), (0, s_dim - len(roots))))
        bases_stack = jnp.stack(b_rows, axis=0)
        comp_stack = jnp.stack(c_rows, axis=0)
        tables[nt] = _run_matmul(x_pad[nt], bases_stack, comp_stack)

    root_pos = {nt: len(_SRC_BLOCKS[nt]) for nt in _N}
    tables2d = {
        nt: tables[nt].reshape(-1, D) for nt in _N
    }

    agg = {}
    for nt in _N:
        dst_ets = _DST_ETS[nt]
        src_nts = sorted({_ETS[e][0] for e in dst_ets}, key=lambda n: _TID[n])
        idx_parts, dst_parts, streams, offs = [], [], [], []
        off = 0
        for slot, src_nt in enumerate(src_nts):
            rows_u, dsts_u = [], []
            for e in dst_ets:
                if _ETS[e][0] != src_nt:
                    continue
                ei = inp["ei_" + e]
                rows_u.append(_POS[e] * _NPAD[src_nt] + ei[0].astype(I32))
                dsts_u.append(ei[1].astype(I32))
            row_u = jnp.concatenate(rows_u)
            dst_u = jnp.concatenate(dsts_u)
            e_s = row_u.shape[0]
            e_pad = _ceil_to(e_s, ALIGN)
            dst_s, row_s = lax.sort_key_val(dst_u, row_u)
            pad_idx = (jnp.arange(e_pad - e_s, dtype=I32) * 8) % 30000
            idx_parts.append(jnp.concatenate([row_s, pad_idx]))
            dst_parts.append(jnp.pad(dst_s, (0, e_pad - e_s),
                                     constant_values=BIG))
            streams.append((slot, off, e_pad // NW, e_pad // ALIGN))
            offs.append(off)
            off += e_pad
        idx_all = jnp.concatenate(idx_parts)

        src_tables = [tables2d[n] for n in src_nts]
        gk = _make_gather(streams, len(src_tables), off)
        msgs = gk(*src_tables, idx_all)

        nb = _NPAD[nt] // BLK
        t_static = off // CHK + nb * len(src_nts)
        eb, ec, fb = _chunk_tables(dst_parts, offs, nb, t_static)
        dst3 = jnp.concatenate(dst_parts).reshape(-1, 1, CHK)
        seg = _make_seg(nb, root_pos[nt], t_static)
        agg[nt] = seg(eb, ec, fb, dst3, msgs, tables[nt])

    res = []
    for nt in _N:
        bsum = sum(inp["bias_" + e] for e in _DST_ETS[nt])
        out = _run_epilogue(agg[nt], x_pad[nt], bsum,
                            inp["ln_gamma_" + nt], inp["ln_beta_" + nt])
        res.append(out[: _N[nt]])
    return tuple(res)
